# baseline (speedup 1.0000x reference)
import functools

import jax
import jax.numpy as jnp
from jax.experimental import pallas as pl
from jax.experimental.pallas import tpu as pltpu

D = 1024
H = 2048
HALF = D // 2


def _flow_body(x_ref, w1_ref, b1_ref, w2_ref, b2_ref, permc_ref, perm_ref,
               y_ref, ld_ref):
    pc = permc_ref[...]
    cols = jax.lax.broadcasted_iota(jnp.int32, (D, HALF), 1)
    G1 = (pc == cols).astype(jnp.float32)
    pm = perm_ref[...]
    jrows = jax.lax.broadcasted_iota(jnp.int32, (HALF, D), 0)
    M = (pm == jrows + HALF).astype(jnp.float32)

    w1 = w1_ref[...]
    w2 = w2_ref[...]
    b1 = b1_ref[...]
    b2 = b2_ref[...]

    def half_chain(xb):
        x1 = jnp.dot(xb, G1, preferred_element_type=jnp.float32)
        h = jnp.tanh(jnp.dot(x1, w1, preferred_element_type=jnp.float32) + b1)
        params = jnp.dot(h, w2, preferred_element_type=jnp.float32) + b2
        s = jnp.tanh(params[:, :HALF])
        t = params[:, HALF:]
        S = jnp.dot(s, M, preferred_element_type=jnp.float32)
        T = jnp.dot(t, M, preferred_element_type=jnp.float32)
        return xb * jnp.exp(S) + T, jnp.sum(s, axis=1, keepdims=True)

    for c in range(4):
        yc, ldc = half_chain(x_ref[c * 256:(c + 1) * 256, :])
        y_ref[c * 256:(c + 1) * 256, :] = yc
        ld_ref[c * 256:(c + 1) * 256, :] = ldc


@functools.partial(jax.jit, static_argnames=("interpret",))
def _run(x, W1, b1, W2, b2, perm_col, perm_2d, interpret=False):
    N = x.shape[0]
    R = 1024
    grid = (N // R,)

    y, ld = pl.pallas_call(
        _flow_body,
        grid=grid,
        in_specs=[
            pl.BlockSpec((R, D), lambda i: (i, 0)),
            pl.BlockSpec((HALF, H), lambda i: (0, 0)),
            pl.BlockSpec((1, H), lambda i: (0, 0)),
            pl.BlockSpec((H, D), lambda i: (0, 0)),
            pl.BlockSpec((1, D), lambda i: (0, 0)),
            pl.BlockSpec((D, 1), lambda i: (0, 0)),
            pl.BlockSpec((1, D), lambda i: (0, 0)),
        ],
        out_specs=[
            pl.BlockSpec((R, D), lambda i: (i, 0)),
            pl.BlockSpec((R, 1), lambda i: (i, 0)),
        ],
        out_shape=[
            jax.ShapeDtypeStruct((N, D), jnp.float32),
            jax.ShapeDtypeStruct((N, 1), jnp.float32),
        ],
        interpret=interpret,
    )(x, W1, b1.reshape(1, H), W2, b2.reshape(1, D), perm_col, perm_2d)
    return y, ld[:, 0]


def kernel(x, W1, b1, W2, b2, perm):
    perm = perm.astype(jnp.int32)
    return _run(x, W1, b1, W2, b2, perm.reshape(D, 1), perm.reshape(1, D))


# speedup vs baseline: 1.0426x; 1.0426x over previous
import functools

import jax
import jax.numpy as jnp
from jax.experimental import pallas as pl
from jax.experimental.pallas import tpu as pltpu

D = 1024
H = 2048
HALF = D // 2


def _flow_body(x_ref, w1_ref, b1_ref, w2_ref, b2_ref, permc_ref, perm_ref,
               y_ref, ld_ref):
    pc = permc_ref[...]
    cols = jax.lax.broadcasted_iota(jnp.int32, (D, HALF), 1)
    G1 = (pc == cols).astype(jnp.float32)
    pm = perm_ref[...]
    jrows = jax.lax.broadcasted_iota(jnp.int32, (HALF, D), 0)
    M = (pm == jrows + HALF).astype(jnp.float32)

    w1 = w1_ref[...]
    w2 = w2_ref[...]
    b1 = b1_ref[...]
    b2 = b2_ref[...]

    def half_chain(xb):
        x1 = jnp.dot(xb, G1, preferred_element_type=jnp.float32)
        h = jnp.tanh(jnp.dot(x1, w1, preferred_element_type=jnp.float32) + b1)
        params = jnp.dot(h, w2, preferred_element_type=jnp.float32) + b2
        s = jnp.tanh(params[:, :HALF])
        t = params[:, HALF:]
        S = jnp.dot(s, M, preferred_element_type=jnp.float32)
        T = jnp.dot(t, M, preferred_element_type=jnp.float32)
        return xb * jnp.exp(S) + T, jnp.sum(s, axis=1, keepdims=True)

    y0, ld0 = half_chain(x_ref[0:512, :])
    y1, ld1 = half_chain(x_ref[512:1024, :])
    y_ref[0:512, :] = y0
    y_ref[512:1024, :] = y1
    ld_ref[0:512, :] = ld0
    ld_ref[512:1024, :] = ld1


@functools.partial(jax.jit, static_argnames=("interpret",))
def _run(x, W1, b1, W2, b2, perm_col, perm_2d, interpret=False):
    N = x.shape[0]
    R = 1024
    grid = (N // R,)

    y, ld = pl.pallas_call(
        _flow_body,
        grid=grid,
        in_specs=[
            pl.BlockSpec((R, D), lambda i: (i, 0)),
            pl.BlockSpec((HALF, H), lambda i: (0, 0)),
            pl.BlockSpec((1, H), lambda i: (0, 0)),
            pl.BlockSpec((H, D), lambda i: (0, 0)),
            pl.BlockSpec((1, D), lambda i: (0, 0)),
            pl.BlockSpec((D, 1), lambda i: (0, 0)),
            pl.BlockSpec((1, D), lambda i: (0, 0)),
        ],
        out_specs=[
            pl.BlockSpec((R, D), lambda i: (i, 0)),
            pl.BlockSpec((R, 1), lambda i: (i, 0)),
        ],
        out_shape=[
            jax.ShapeDtypeStruct((N, D), jnp.float32),
            jax.ShapeDtypeStruct((N, 1), jnp.float32),
        ],
        interpret=interpret,
    )(x, W1, b1.reshape(1, H), W2, b2.reshape(1, D), perm_col, perm_2d)
    return y, ld[:, 0]


def kernel(x, W1, b1, W2, b2, perm):
    perm = perm.astype(jnp.int32)
    return _run(x, W1, b1, W2, b2, perm.reshape(D, 1), perm.reshape(1, D))
